# baseline (speedup 1.0000x reference)
import functools

import jax
import jax.numpy as jnp
from jax.experimental import pallas as pl

_NUM_EMBEDDINGS = 512
_DIM = 32
_COMMITMENT_COST = 0.25
_BLOCK = 4000


def _vq_block_kernel(h_ref, cb_ref, cc_ref, q_ref, idx_ref, loss_ref):
    h = h_ref[...]
    cb = cb_ref[...]
    hh = jnp.sum(h * h, axis=1, keepdims=True)
    cc = cc_ref[0, :]
    cross2 = jax.lax.dot_general(
        h * (-2.0), cb, (((1,), (1,)), ((), ())),
        preferred_element_type=jnp.float32)
    d = (hh + cc[None, :]) + cross2
    dmin = jnp.min(d, axis=1, keepdims=True)
    iota_f = jax.lax.broadcasted_iota(jnp.int32, d.shape, 1).astype(jnp.float32)
    idx_f = jnp.min(jnp.where(d <= dmin, iota_f, float(_NUM_EMBEDDINGS)),
                    axis=1, keepdims=True)
    idx = idx_f.astype(jnp.int32)[:, 0]
    onehot = (iota_f == idx_f).astype(jnp.bfloat16)
    q = jax.lax.dot_general(
        onehot, cb.astype(jnp.bfloat16), (((1,), (0,)), ((), ())),
        preferred_element_type=jnp.float32)
    q_ref[...] = q
    idx_ref[...] = idx[:, None]
    part = jnp.sum(dmin, axis=0, keepdims=True)
    prev = jnp.where(pl.program_id(0) == 0, 0.0, loss_ref[...])
    loss_ref[...] = prev + part


@functools.partial(jax.jit, static_argnames=())
def kernel(h_v_k, codebook):
    n, d = h_v_k.shape
    e = codebook.shape[0]
    cc = jnp.sum(codebook * codebook, axis=1)[None, :]
    grid = n // _BLOCK
    q, idx, loss = pl.pallas_call(
        _vq_block_kernel,
        grid=(grid,),
        in_specs=[
            pl.BlockSpec((_BLOCK, d), lambda i: (i, 0)),
            pl.BlockSpec((e, d), lambda i: (0, 0)),
            pl.BlockSpec((1, e), lambda i: (0, 0)),
        ],
        out_specs=[
            pl.BlockSpec((_BLOCK, d), lambda i: (i, 0)),
            pl.BlockSpec((_BLOCK, 1), lambda i: (i, 0)),
            pl.BlockSpec((1, 1), lambda i: (0, 0)),
        ],
        out_shape=[
            jax.ShapeDtypeStruct((n, d), jnp.float32),
            jax.ShapeDtypeStruct((n, 1), jnp.int32),
            jax.ShapeDtypeStruct((1, 1), jnp.float32),
        ],
    )(h_v_k, codebook, cc)
    total_loss = loss[0, 0] * ((1.0 + _COMMITMENT_COST) / (n * d))
    return (q, idx.reshape(n), total_loss)


# speedup vs baseline: 1.1012x; 1.1012x over previous
import functools

import jax
import jax.numpy as jnp
from jax.experimental import pallas as pl
from jax.experimental.pallas import tpu as pltpu

_NUM_EMBEDDINGS = 512
_DIM = 32
_COMMITMENT_COST = 0.25
_BLOCK = 4000


def _vq_block_kernel(h_ref, cb_ref, cc_ref, q_ref, idx_ref, loss_ref):
    h = h_ref[...]
    cb = cb_ref[...]
    hh = jnp.sum(h * h, axis=1, keepdims=True)
    cc = cc_ref[0, :]
    cross2 = jax.lax.dot_general(
        h * (-2.0), cb, (((1,), (1,)), ((), ())),
        preferred_element_type=jnp.float32)
    d = (hh + cc[None, :]) + cross2
    dmin = jnp.min(d, axis=1, keepdims=True)
    iota_f = jax.lax.broadcasted_iota(jnp.int32, d.shape, 1).astype(jnp.float32)
    idx_f = jnp.min(jnp.where(d <= dmin, iota_f, float(_NUM_EMBEDDINGS)),
                    axis=1, keepdims=True)
    onehot = (iota_f == idx_f).astype(jnp.float32)
    q = jax.lax.dot_general(
        onehot, cb, (((1,), (0,)), ((), ())),
        preferred_element_type=jnp.float32)
    q_ref[...] = q
    b = idx_f.shape[0]
    idx_ref[...] = jnp.reshape(idx_f.astype(jnp.int32), (1, 8, b // 8))
    loss_ref[...] = jnp.sum(dmin, axis=0, keepdims=True)[None]


@functools.partial(jax.jit, static_argnames=())
def kernel(h_v_k, codebook):
    n, d = h_v_k.shape
    e = codebook.shape[0]
    cc = jnp.sum(codebook * codebook, axis=1)[None, :]
    grid = n // _BLOCK
    q, idx, loss = pl.pallas_call(
        _vq_block_kernel,
        grid=(grid,),
        in_specs=[
            pl.BlockSpec((_BLOCK, d), lambda i: (i, 0)),
            pl.BlockSpec((e, d), lambda i: (0, 0)),
            pl.BlockSpec((1, e), lambda i: (0, 0)),
        ],
        out_specs=[
            pl.BlockSpec((_BLOCK, d), lambda i: (i, 0)),
            pl.BlockSpec((1, 8, _BLOCK // 8), lambda i: (i, 0, 0)),
            pl.BlockSpec((1, 1, 1), lambda i: (i, 0, 0)),
        ],
        out_shape=[
            jax.ShapeDtypeStruct((n, d), jnp.float32),
            jax.ShapeDtypeStruct((grid, 8, _BLOCK // 8), jnp.int32),
            jax.ShapeDtypeStruct((grid, 1, 1), jnp.float32),
        ],
        compiler_params=pltpu.CompilerParams(
            dimension_semantics=("parallel",)),
    )(h_v_k, codebook, cc)
    total_loss = jnp.sum(loss) * ((1.0 + _COMMITMENT_COST) / (n * d))
    return (q, idx.reshape(n), total_loss)
